# baseline (speedup 1.0000x reference)
import functools
import math

import jax
import jax.numpy as jnp
from jax import lax
from jax.experimental import pallas as pl
from jax.experimental.pallas import tpu as pltpu
from jax.experimental.pallas import tpu_sc as plsc


_TT = 2048


def _score_body(x_ref, w_ref, o_ref):
    o_ref[0] = lax.dot_general(
        w_ref[...], x_ref[...], (((1,), (1,)), ((), ())),
        preferred_element_type=jnp.float32)


def _scores(x2, w2r):
    nbt = x2.shape[0] // _TT
    d = x2.shape[1]
    return pl.pallas_call(
        _score_body,
        grid=(nbt,),
        in_specs=[
            pl.BlockSpec((_TT, d), lambda i: (i, 0)),
            pl.BlockSpec((1, d), lambda i: (0, 0)),
        ],
        out_specs=pl.BlockSpec((1, 1, _TT), lambda i: (i, 0, 0)),
        out_shape=jax.ShapeDtypeStruct((nbt, 1, _TT), jnp.float32),
    )(x2, w2r)


_RC = 512
_LB = 64


def _sortable(v):
    u = lax.bitcast_convert_type(v, jnp.int32)
    return u ^ (lax.shift_right_arithmetic(u, 31) & jnp.int32(0x7FFFFFFF))


def _topk_body(srow_ref, idx_ref, gidx_ref):
    b_sz, t = srow_ref.shape
    kh = idx_ref.shape[1]
    ones_col = jnp.ones((t, 1), jnp.float32)
    iota_col = lax.broadcasted_iota(jnp.int32, (t, 1), 0).astype(jnp.float32)
    hi_iota = lax.broadcasted_iota(jnp.int32, (kh, 1), 0)
    lo_iota = lax.broadcasted_iota(jnp.int32, (1, _LB), 1)
    jl_diag = (lax.broadcasted_iota(jnp.int32, (_RC, _RC), 1)
               < lax.broadcasted_iota(jnp.int32, (_RC, _RC), 0))
    for b in range(b_sz):
        ks_row = _sortable(srow_ref[b:b + 1, :])
        ks_col = jnp.transpose(ks_row, (1, 0))
        acc_blocks = []
        for ic in range(t // _RC):
            lo, hi = ic * _RC, (ic + 1) * _RC
            ks_i = ks_col[lo:hi, :]
            a = jnp.zeros((_RC, 1), jnp.float32)
            if lo > 0:
                ge = (ks_row[:, :lo] >= ks_i).astype(jnp.float32)
                a = a + jnp.dot(ge, ones_col[:lo, :],
                                preferred_element_type=jnp.float32)
            ksd = ks_row[:, lo:hi]
            diag = ((ksd > ks_i) | ((ksd == ks_i) & jl_diag)
                    ).astype(jnp.float32)
            a = a + jnp.dot(diag, ones_col[:_RC, :],
                            preferred_element_type=jnp.float32)
            if hi < t:
                gt = (ks_row[:, hi:] > ks_i).astype(jnp.float32)
                a = a + jnp.dot(gt, ones_col[:t - hi, :],
                                preferred_element_type=jnp.float32)
            acc_blocks.append(a)
        rank = jnp.concatenate(acc_blocks, axis=0).astype(jnp.int32)
        rank_row = jnp.transpose(rank, (1, 0))
        h_t = (hi_iota == lax.shift_right_logical(rank_row, 6)
               ).astype(jnp.float32)
        l_m = ((rank & jnp.int32(_LB - 1)) == lo_iota
               ).astype(jnp.float32)
        il = l_m * iota_col
        a_idx = jnp.dot(h_t, il,
                        preferred_element_type=jnp.float32)
        ai = a_idx.astype(jnp.int32)
        idx_ref[b] = ai
        gidx_ref[b] = ai + b * t


def _topk(srow, k):
    b, t = srow.shape
    kh = k // _LB
    return pl.pallas_call(
        _topk_body,
        out_shape=(
            jax.ShapeDtypeStruct((b, kh, _LB), jnp.int32),
            jax.ShapeDtypeStruct((b, kh, _LB), jnp.int32),
        ),
    )(srow)


_CH = 16


def _make_sc_gather(n_rows, d):
    info = plsc.get_sparse_core_info()
    nw = info.num_cores * info.num_subcores
    nc = info.num_cores
    b_per_w = n_rows // nw
    n_ch = b_per_w // _CH
    mesh = plsc.VectorSubcoreMesh(core_axis_name="c", subcore_axis_name="s")

    @functools.partial(
        pl.kernel,
        mesh=mesh,
        out_type=jax.ShapeDtypeStruct((n_rows, d), jnp.float32),
        scratch_types=[
            pltpu.VMEM((b_per_w,), jnp.int32),
            pltpu.VMEM((_CH,), jnp.int32),
            pltpu.VMEM((_CH,), jnp.int32),
            pltpu.VMEM((_CH,), jnp.int32),
            pltpu.VMEM((_CH, d), jnp.float32),
            pltpu.VMEM((_CH, d), jnp.float32),
            pltpu.VMEM((_CH, d), jnp.float32),
            pltpu.SemaphoreType.DMA,
            pltpu.SemaphoreType.DMA,
            pltpu.SemaphoreType.DMA,
            pltpu.SemaphoreType.DMA,
            pltpu.SemaphoreType.DMA,
            pltpu.SemaphoreType.DMA,
        ],
    )
    def gather_k(table_hbm, idx_hbm, out_hbm,
                 idx_all, i16_0, i16_1, i16_2, rows_v0, rows_v1, rows_v2,
                 sem_g0, sem_g1, sem_g2, sem_o0, sem_o1, sem_o2):
        wid = lax.axis_index("s") * nc + lax.axis_index("c")
        base = wid * b_per_w
        pltpu.sync_copy(idx_hbm.at[pl.ds(base, b_per_w)], idx_all)
        idx16 = [i16_0, i16_1, i16_2]
        rows_v = [rows_v0, rows_v1, rows_v2]
        sem_g = [sem_g0, sem_g1, sem_g2]
        sem_o = [sem_o0, sem_o1, sem_o2]

        def start_gather(c):
            p = c % 3
            idx16[p][...] = idx_all[pl.ds(c * _CH, _CH)]
            return pltpu.async_copy(table_hbm.at[idx16[p]], rows_v[p],
                                    sem_g[p])

        g = [None] * n_ch
        w = [None] * n_ch
        g[0] = start_gather(0)
        if n_ch > 1:
            g[1] = start_gather(1)
        for c in range(n_ch):
            p = c % 3
            g[c].wait()
            w[c] = pltpu.async_copy(
                rows_v[p], out_hbm.at[pl.ds(base + c * _CH, _CH)], sem_o[p])
            if c + 2 < n_ch:
                if c >= 1:
                    w[c - 1].wait()
                g[c + 2] = start_gather(c + 2)
        for c in range(max(0, n_ch - 3), n_ch):
            if w[c] is not None:
                w[c].wait()

    return gather_k


def kernel(x, W):
    b, t, d = x.shape
    k = max(1, math.ceil(0.5 * t))

    x2 = x.reshape(b * t, d)
    srow = _scores(x2, W.reshape(1, d)).reshape(b, t)
    indices, gidx = _topk(srow, k)

    gather_fn = _make_sc_gather(b * k, d)
    selected = gather_fn(x2, gidx.reshape(b * k))
    return selected.reshape(b, k, d), indices.reshape(b, k), srow


# speedup vs baseline: 1.0491x; 1.0491x over previous
import functools
import math

import jax
import jax.numpy as jnp
from jax import lax
from jax.experimental import pallas as pl
from jax.experimental.pallas import tpu as pltpu
from jax.experimental.pallas import tpu_sc as plsc


_TT = 2048


def _score_body(x_ref, w_ref, o_ref):
    o_ref[0] = lax.dot_general(
        w_ref[...], x_ref[...], (((1,), (1,)), ((), ())),
        preferred_element_type=jnp.float32)


def _scores(x2, w2r):
    nbt = x2.shape[0] // _TT
    d = x2.shape[1]
    return pl.pallas_call(
        _score_body,
        grid=(nbt,),
        in_specs=[
            pl.BlockSpec((_TT, d), lambda i: (i, 0)),
            pl.BlockSpec((1, d), lambda i: (0, 0)),
        ],
        out_specs=pl.BlockSpec((1, 1, _TT), lambda i: (i, 0, 0)),
        out_shape=jax.ShapeDtypeStruct((nbt, 1, _TT), jnp.float32),
    )(x2, w2r)


_RC = 512
_LB = 64


def _sortable(v):
    u = lax.bitcast_convert_type(v, jnp.int32)
    return u ^ (lax.shift_right_arithmetic(u, 31) & jnp.int32(0x7FFFFFFF))


def _topk_body(srow_ref, idx_ref, gidx_ref):
    b_sz, t = srow_ref.shape
    kh = idx_ref.shape[1]
    ones_col = jnp.ones((t, 1), jnp.float32)
    iota_col = lax.broadcasted_iota(jnp.int32, (t, 1), 0).astype(jnp.float32)
    hi_iota = lax.broadcasted_iota(jnp.int32, (kh, 1), 0)
    lo_iota = lax.broadcasted_iota(jnp.int32, (1, _LB), 1)
    jl_diag = (lax.broadcasted_iota(jnp.int32, (_RC, _RC), 1)
               < lax.broadcasted_iota(jnp.int32, (_RC, _RC), 0))
    ones_row = jnp.ones((1, _RC), jnp.float32)
    const_row = (lax.broadcasted_iota(jnp.int32, (1, srow_ref.shape[1]), 1)
                 & jnp.int32(~(_RC - 1))).astype(jnp.float32)
    for b in range(b_sz):
        ks_row = _sortable(srow_ref[b:b + 1, :])
        ks_col = jnp.transpose(ks_row, (1, 0))
        acc_blocks = []
        row_acc = const_row
        for ic in range(t // _RC):
            lo, hi = ic * _RC, (ic + 1) * _RC
            ks_i = ks_col[lo:hi, :]
            ksd = ks_row[:, lo:hi]
            diag = ((ksd > ks_i) | ((ksd == ks_i) & jl_diag)
                    ).astype(jnp.float32)
            a = jnp.dot(diag, ones_col[:_RC, :],
                        preferred_element_type=jnp.float32)
            if hi < t:
                m = (ks_row[:, hi:] > ks_i).astype(jnp.float32)
                a = a + jnp.dot(m, ones_col[:t - hi, :],
                                preferred_element_type=jnp.float32)
                cs = jnp.dot(ones_row, m,
                             preferred_element_type=jnp.float32)
                row_acc = jnp.concatenate(
                    [row_acc[:, :hi], row_acc[:, hi:] - cs], axis=1)
            acc_blocks.append(a)
        rank = (jnp.concatenate(acc_blocks, axis=0)
                + jnp.transpose(row_acc, (1, 0))).astype(jnp.int32)
        rank_row = jnp.transpose(rank, (1, 0))
        h_t = (hi_iota == lax.shift_right_logical(rank_row, 6)
               ).astype(jnp.float32)
        l_m = ((rank & jnp.int32(_LB - 1)) == lo_iota
               ).astype(jnp.float32)
        il = l_m * iota_col
        a_idx = jnp.dot(h_t, il,
                        preferred_element_type=jnp.float32)
        ai = a_idx.astype(jnp.int32)
        idx_ref[b] = ai
        gidx_ref[b] = ai + b * t


def _topk(srow, k):
    b, t = srow.shape
    kh = k // _LB
    return pl.pallas_call(
        _topk_body,
        out_shape=(
            jax.ShapeDtypeStruct((b, kh, _LB), jnp.int32),
            jax.ShapeDtypeStruct((b, kh, _LB), jnp.int32),
        ),
    )(srow)


_CH = 16


def _make_sc_gather(n_rows, d):
    info = plsc.get_sparse_core_info()
    nw = info.num_cores * info.num_subcores
    nc = info.num_cores
    b_per_w = n_rows // nw
    n_ch = b_per_w // _CH
    mesh = plsc.VectorSubcoreMesh(core_axis_name="c", subcore_axis_name="s")

    @functools.partial(
        pl.kernel,
        mesh=mesh,
        out_type=jax.ShapeDtypeStruct((n_rows, d), jnp.float32),
        scratch_types=[
            pltpu.VMEM((b_per_w,), jnp.int32),
            pltpu.VMEM((_CH,), jnp.int32),
            pltpu.VMEM((_CH,), jnp.int32),
            pltpu.VMEM((_CH,), jnp.int32),
            pltpu.VMEM((_CH, d), jnp.float32),
            pltpu.VMEM((_CH, d), jnp.float32),
            pltpu.VMEM((_CH, d), jnp.float32),
            pltpu.SemaphoreType.DMA,
            pltpu.SemaphoreType.DMA,
            pltpu.SemaphoreType.DMA,
            pltpu.SemaphoreType.DMA,
            pltpu.SemaphoreType.DMA,
            pltpu.SemaphoreType.DMA,
        ],
    )
    def gather_k(table_hbm, idx_hbm, out_hbm,
                 idx_all, i16_0, i16_1, i16_2, rows_v0, rows_v1, rows_v2,
                 sem_g0, sem_g1, sem_g2, sem_o0, sem_o1, sem_o2):
        wid = lax.axis_index("s") * nc + lax.axis_index("c")
        base = wid * b_per_w
        pltpu.sync_copy(idx_hbm.at[pl.ds(base, b_per_w)], idx_all)
        idx16 = [i16_0, i16_1, i16_2]
        rows_v = [rows_v0, rows_v1, rows_v2]
        sem_g = [sem_g0, sem_g1, sem_g2]
        sem_o = [sem_o0, sem_o1, sem_o2]

        def start_gather(c):
            p = c % 3
            idx16[p][...] = idx_all[pl.ds(c * _CH, _CH)]
            return pltpu.async_copy(table_hbm.at[idx16[p]], rows_v[p],
                                    sem_g[p])

        g = [None] * n_ch
        w = [None] * n_ch
        g[0] = start_gather(0)
        if n_ch > 1:
            g[1] = start_gather(1)
        for c in range(n_ch):
            p = c % 3
            g[c].wait()
            w[c] = pltpu.async_copy(
                rows_v[p], out_hbm.at[pl.ds(base + c * _CH, _CH)], sem_o[p])
            if c + 2 < n_ch:
                if c >= 1:
                    w[c - 1].wait()
                g[c + 2] = start_gather(c + 2)
        for c in range(max(0, n_ch - 3), n_ch):
            if w[c] is not None:
                w[c].wait()

    return gather_k


def kernel(x, W):
    b, t, d = x.shape
    k = max(1, math.ceil(0.5 * t))

    x2 = x.reshape(b * t, d)
    srow = _scores(x2, W.reshape(1, d)).reshape(b, t)
    indices, gidx = _topk(srow, k)

    gather_fn = _make_sc_gather(b * k, d)
    selected = gather_fn(x2, gidx.reshape(b * k))
    return selected.reshape(b, k, d), indices.reshape(b, k), srow
